# baseline (speedup 1.0000x reference)
import jax
import jax.numpy as jnp
from jax import lax
from jax.experimental import pallas as pl
from jax.experimental.pallas import tpu as pltpu
from jax.experimental.pallas import tpu_sc as plsc

N_PRED = 64
N_CLAUSES = 32
ROW_PITCH = N_PRED + 1

CHUNK_ROWS = 160
TILES_PER_CHUNK = CHUNK_ROWS // 16
N_WORKERS = 32


def _sc_body(ga, ew, cw2, out, in_bufs, ew_bufs, cw_buf, out_bufs, in_sems, out_sems):
    cid = lax.axis_index("c")
    sid = lax.axis_index("s")
    w = sid * 2 + cid
    pltpu.sync_copy(cw2, cw_buf)
    nchunks = ga.shape[0] // CHUNK_ROWS
    n_my = (nchunks - 1 - w) // N_WORKERS + 1
    iota16 = lax.iota(jnp.int32, 16)

    def row0_of(i):
        return (w + i * N_WORKERS) * CHUNK_ROWS

    def start_in(i, b):
        row0 = row0_of(i)
        pltpu.async_copy(
            ga.at[pl.ds(row0, CHUNK_ROWS)], in_bufs[b].at[:, pl.ds(0, N_PRED)],
            in_sems[b])
        pltpu.async_copy(ew.at[pl.ds(row0, CHUNK_ROWS)], ew_bufs[b], in_sems[b])

    def wait_in(b):
        pltpu.make_async_copy(
            ga.at[pl.ds(0, CHUNK_ROWS)], in_bufs[b].at[:, pl.ds(0, N_PRED)],
            in_sems[b]).wait()
        pltpu.make_async_copy(ew.at[pl.ds(0, CHUNK_ROWS)], ew_bufs[b],
                              in_sems[b]).wait()

    def start_out(i, b):
        pltpu.async_copy(
            out_bufs[b].at[:, pl.ds(0, N_PRED)],
            out.at[pl.ds(row0_of(i), CHUNK_ROWS)], out_sems[b])

    def wait_out(b):
        pltpu.make_async_copy(
            out_bufs[b].at[:, pl.ds(0, N_PRED)],
            out.at[pl.ds(0, CHUNK_ROWS)], out_sems[b]).wait()

    def compute(b):
        in_buf = in_bufs[b]
        out_buf = out_bufs[b]
        ew_buf = ew_bufs[b]

        @plsc.parallel_loop(0, TILES_PER_CHUNK)
        def tile_body(t):
            r0 = t * 16
            rows = r0 + iota16
            ewv = ew_buf[pl.ds(r0, 16)]

            def clause_deltas(c, c0, c1, c2):
                v0 = plsc.load_gather(in_buf, [rows, c0])
                v1 = plsc.load_gather(in_buf, [rows, c1])
                v2 = plsc.load_gather(in_buf, [rows, c2])
                a = jnp.exp(-(v0 + v1))
                b = jnp.exp(v2 - v1)
                s = cw_buf[c, :] * ewv / (a + 1.0 + b)
                return -(a * s), s, b * s

            @plsc.parallel_loop(0, 21, unroll=7,
                                carry=jnp.zeros((16,), jnp.int32))
            def first_loop(c, col):
                d0, d1, d2 = clause_deltas(c, col, col + 1, col + 2)
                plsc.store_scatter(out_buf, [rows, col], d0)
                plsc.store_scatter(out_buf, [rows, col + 1], d1)
                plsc.store_scatter(out_buf, [rows, col + 2], d2)
                return col + 3

            c63 = jnp.full((16,), 63, jnp.int32)
            c_0 = jnp.zeros((16,), jnp.int32)
            d0, d1, d2 = clause_deltas(21, c63, c_0, c_0 + 1)
            plsc.store_scatter(out_buf, [rows, c63], d0)
            plsc.addupdate_scatter(out_buf, [rows, c_0], d1)
            plsc.addupdate_scatter(out_buf, [rows, c_0 + 1], d2)

            @plsc.parallel_loop(22, N_CLAUSES, unroll=5,
                                carry=jnp.full((16,), 2, jnp.int32))
            def second_loop(c, col):
                d0, d1, d2 = clause_deltas(c, col, col + 1, col + 2)
                plsc.addupdate_scatter(out_buf, [rows, col], d0)
                plsc.addupdate_scatter(out_buf, [rows, col + 1], d1)
                plsc.addupdate_scatter(out_buf, [rows, col + 2], d2)
                return col + 3

    start_in(0, 0)

    @pl.when(n_my > 1)
    def _():
        start_in(1, 1)

    def pair_body(ip, carry):
        for b in range(2):
            i = ip * 2 + b

            @pl.when(i < n_my)
            def _():
                wait_in(b)

                @pl.when(i >= 2)
                def _():
                    wait_out(b)

                compute(b)
                start_out(i, b)

                @pl.when(i + 2 < n_my)
                def _():
                    start_in(i + 2, b)

        return carry

    lax.fori_loop(0, (n_my + 1) // 2, pair_body, 0)

    @pl.when(n_my >= 1)
    def _():
        wait_out(0)

    @pl.when(n_my >= 2)
    def _():
        wait_out(1)


def kernel(ground_atoms, edge_weight, clause_weights):
    b, n_pred = ground_atoms.shape
    assert n_pred == N_PRED and b % CHUNK_ROWS == 0
    ew = edge_weight.reshape(-1)
    cw2 = jnp.broadcast_to(clause_weights[:, None], (N_CLAUSES, 16))
    mesh = plsc.VectorSubcoreMesh(
        core_axis_name="c", subcore_axis_name="s", num_cores=2, num_subcores=16
    )
    call = pl.kernel(
        _sc_body,
        out_type=jax.ShapeDtypeStruct((b, N_PRED), jnp.float32),
        mesh=mesh,
        scratch_types=[
            [pltpu.VMEM((CHUNK_ROWS, ROW_PITCH), jnp.float32) for _ in range(2)],
            [pltpu.VMEM((CHUNK_ROWS,), jnp.float32) for _ in range(2)],
            pltpu.VMEM((N_CLAUSES, 16), jnp.float32),
            [pltpu.VMEM((CHUNK_ROWS, ROW_PITCH), jnp.float32) for _ in range(2)],
            [pltpu.SemaphoreType.DMA for _ in range(2)],
            [pltpu.SemaphoreType.DMA for _ in range(2)],
        ],
        compiler_params=pltpu.CompilerParams(
            use_tc_tiling_on_sc=False, needs_layout_passes=False
        ),
    )
    return call(ground_atoms, ew, cw2)


# speedup vs baseline: 1.0705x; 1.0705x over previous
import jax
import jax.numpy as jnp
from jax import lax
from jax.experimental import pallas as pl
from jax.experimental.pallas import tpu as pltpu
from jax.experimental.pallas import tpu_sc as plsc

N_PRED = 64
N_CLAUSES = 32
ROW_PITCH = N_PRED + 1

CHUNK_ROWS = 160
TILES_PER_CHUNK = CHUNK_ROWS // 16
N_WORKERS = 32


def _sc_body(ga, ew, cw2, out, in_bufs, ew_bufs, cw_buf, out_bufs, in_sems, out_sems):
    cid = lax.axis_index("c")
    sid = lax.axis_index("s")
    w = sid * 2 + cid
    pltpu.sync_copy(cw2, cw_buf)
    nchunks = ga.shape[0] // CHUNK_ROWS
    n_my = (nchunks - 1 - w) // N_WORKERS + 1
    iota16 = lax.iota(jnp.int32, 16)

    def row0_of(i):
        return (w + i * N_WORKERS) * CHUNK_ROWS

    def start_in(i, b):
        row0 = row0_of(i)
        pltpu.async_copy(
            ga.at[pl.ds(row0, CHUNK_ROWS)], in_bufs[b].at[:, pl.ds(0, N_PRED)],
            in_sems[b])
        pltpu.async_copy(ew.at[pl.ds(row0, CHUNK_ROWS)], ew_bufs[b], in_sems[b])

    def wait_in(b):
        pltpu.make_async_copy(
            ga.at[pl.ds(0, CHUNK_ROWS)], in_bufs[b].at[:, pl.ds(0, N_PRED)],
            in_sems[b]).wait()
        pltpu.make_async_copy(ew.at[pl.ds(0, CHUNK_ROWS)], ew_bufs[b],
                              in_sems[b]).wait()

    def start_out(i, b):
        pltpu.async_copy(
            out_bufs[b].at[:, pl.ds(0, N_PRED)],
            out.at[pl.ds(row0_of(i), CHUNK_ROWS)], out_sems[b])

    def wait_out(b):
        pltpu.make_async_copy(
            out_bufs[b].at[:, pl.ds(0, N_PRED)],
            out.at[pl.ds(0, CHUNK_ROWS)], out_sems[b]).wait()

    def compute(b):
        in_buf = in_bufs[b]
        out_buf = out_bufs[b]
        ew_buf = ew_bufs[b]

        @plsc.parallel_loop(0, TILES_PER_CHUNK)
        def tile_body(t):
            r0 = t * 16
            rows = r0 + iota16
            ewv = ew_buf[pl.ds(r0, 16)]

            def clause_deltas(c, c0, c1, c2):
                v0 = plsc.load_gather(in_buf, [rows, c0])
                v1 = plsc.load_gather(in_buf, [rows, c1])
                v2 = plsc.load_gather(in_buf, [rows, c2])
                a = jnp.exp(-(v0 + v1))
                b = jnp.exp(v2 - v1)
                s = cw_buf[c, :] * ewv / (a + 1.0 + b)
                return -(a * s), s, b * s

            @plsc.parallel_loop(0, 21, unroll=4,
                                carry=jnp.zeros((16,), jnp.int32))
            def first_loop(c, col):
                d0, d1, d2 = clause_deltas(c, col, col + 1, col + 2)
                plsc.store_scatter(out_buf, [rows, col], d0)
                plsc.store_scatter(out_buf, [rows, col + 1], d1)
                plsc.store_scatter(out_buf, [rows, col + 2], d2)
                return col + 3

            c63 = jnp.full((16,), 63, jnp.int32)
            c_0 = jnp.zeros((16,), jnp.int32)
            d0, d1, d2 = clause_deltas(21, c63, c_0, c_0 + 1)
            plsc.store_scatter(out_buf, [rows, c63], d0)
            plsc.addupdate_scatter(out_buf, [rows, c_0], d1)
            plsc.addupdate_scatter(out_buf, [rows, c_0 + 1], d2)

            @plsc.parallel_loop(22, N_CLAUSES, unroll=2,
                                carry=jnp.full((16,), 2, jnp.int32))
            def second_loop(c, col):
                d0, d1, d2 = clause_deltas(c, col, col + 1, col + 2)
                plsc.addupdate_scatter(out_buf, [rows, col], d0)
                plsc.addupdate_scatter(out_buf, [rows, col + 1], d1)
                plsc.addupdate_scatter(out_buf, [rows, col + 2], d2)
                return col + 3

    start_in(0, 0)

    @pl.when(n_my > 1)
    def _():
        start_in(1, 1)

    def pair_body(ip, carry):
        for b in range(2):
            i = ip * 2 + b

            @pl.when(i < n_my)
            def _():
                wait_in(b)

                @pl.when(i >= 2)
                def _():
                    wait_out(b)

                compute(b)
                start_out(i, b)

                @pl.when(i + 2 < n_my)
                def _():
                    start_in(i + 2, b)

        return carry

    lax.fori_loop(0, (n_my + 1) // 2, pair_body, 0)

    @pl.when(n_my >= 1)
    def _():
        wait_out(0)

    @pl.when(n_my >= 2)
    def _():
        wait_out(1)


def kernel(ground_atoms, edge_weight, clause_weights):
    b, n_pred = ground_atoms.shape
    assert n_pred == N_PRED and b % CHUNK_ROWS == 0
    ew = edge_weight.reshape(-1)
    cw2 = jnp.broadcast_to(clause_weights[:, None], (N_CLAUSES, 16))
    mesh = plsc.VectorSubcoreMesh(
        core_axis_name="c", subcore_axis_name="s", num_cores=2, num_subcores=16
    )
    call = pl.kernel(
        _sc_body,
        out_type=jax.ShapeDtypeStruct((b, N_PRED), jnp.float32),
        mesh=mesh,
        scratch_types=[
            [pltpu.VMEM((CHUNK_ROWS, ROW_PITCH), jnp.float32) for _ in range(2)],
            [pltpu.VMEM((CHUNK_ROWS,), jnp.float32) for _ in range(2)],
            pltpu.VMEM((N_CLAUSES, 16), jnp.float32),
            [pltpu.VMEM((CHUNK_ROWS, ROW_PITCH), jnp.float32) for _ in range(2)],
            [pltpu.SemaphoreType.DMA for _ in range(2)],
            [pltpu.SemaphoreType.DMA for _ in range(2)],
        ],
        compiler_params=pltpu.CompilerParams(
            use_tc_tiling_on_sc=False, needs_layout_passes=False
        ),
    )
    return call(ground_atoms, ew, cw2)


# speedup vs baseline: 1.1108x; 1.0376x over previous
import jax
import jax.numpy as jnp
from jax import lax
from jax.experimental import pallas as pl
from jax.experimental.pallas import tpu as pltpu
from jax.experimental.pallas import tpu_sc as plsc

N_PRED = 64
N_CLAUSES = 32
ROW_PITCH = N_PRED + 1

CHUNK_ROWS = 160
TILES_PER_CHUNK = CHUNK_ROWS // 16
N_WORKERS = 32


def _sc_body(ga, ew, cw2, out, in_bufs, ew_bufs, cw_buf, out_bufs, in_sems, out_sems):
    cid = lax.axis_index("c")
    sid = lax.axis_index("s")
    w = sid * 2 + cid
    pltpu.sync_copy(cw2, cw_buf)
    nchunks = ga.shape[0] // CHUNK_ROWS
    n_my = (nchunks - 1 - w) // N_WORKERS + 1
    iota16 = lax.iota(jnp.int32, 16)

    def row0_of(i):
        return (w + i * N_WORKERS) * CHUNK_ROWS

    def start_in(i, b):
        row0 = row0_of(i)
        pltpu.async_copy(
            ga.at[pl.ds(row0, CHUNK_ROWS)], in_bufs[b].at[:, pl.ds(0, N_PRED)],
            in_sems[b])
        pltpu.async_copy(ew.at[pl.ds(row0, CHUNK_ROWS)], ew_bufs[b], in_sems[b])

    def wait_in(b):
        pltpu.make_async_copy(
            ga.at[pl.ds(0, CHUNK_ROWS)], in_bufs[b].at[:, pl.ds(0, N_PRED)],
            in_sems[b]).wait()
        pltpu.make_async_copy(ew.at[pl.ds(0, CHUNK_ROWS)], ew_bufs[b],
                              in_sems[b]).wait()

    def start_out(i, b):
        pltpu.async_copy(
            out_bufs[b].at[:, pl.ds(0, N_PRED)],
            out.at[pl.ds(row0_of(i), CHUNK_ROWS)], out_sems[b])

    def wait_out(b):
        pltpu.make_async_copy(
            out_bufs[b].at[:, pl.ds(0, N_PRED)],
            out.at[pl.ds(0, CHUNK_ROWS)], out_sems[b]).wait()

    def compute(b):
        in_buf = in_bufs[b]
        out_buf = out_bufs[b]
        ew_buf = ew_bufs[b]

        @plsc.parallel_loop(0, TILES_PER_CHUNK)
        def tile_body(t):
            r0 = t * 16
            rows = r0 + iota16
            ewv = ew_buf[pl.ds(r0, 16)]

            def clause_deltas(c, c0, c1, c2):
                v0 = plsc.load_gather(in_buf, [rows, c0])
                v1 = plsc.load_gather(in_buf, [rows, c1])
                v2 = plsc.load_gather(in_buf, [rows, c2])
                a = jnp.exp(-(v0 + v1))
                b = jnp.exp(v2 - v1)
                s = cw_buf[c, :] * ewv / (a + 1.0 + b)
                return -(a * s), s, b * s

            @plsc.parallel_loop(0, 21, unroll=3,
                                carry=jnp.zeros((16,), jnp.int32))
            def first_loop(c, col):
                d0, d1, d2 = clause_deltas(c, col, col + 1, col + 2)
                plsc.store_scatter(out_buf, [rows, col], d0)
                plsc.store_scatter(out_buf, [rows, col + 1], d1)
                plsc.store_scatter(out_buf, [rows, col + 2], d2)
                return col + 3

            c63 = jnp.full((16,), 63, jnp.int32)
            c_0 = jnp.zeros((16,), jnp.int32)
            d0, d1, d2 = clause_deltas(21, c63, c_0, c_0 + 1)
            plsc.store_scatter(out_buf, [rows, c63], d0)
            plsc.addupdate_scatter(out_buf, [rows, c_0], d1)
            plsc.addupdate_scatter(out_buf, [rows, c_0 + 1], d2)

            @plsc.parallel_loop(22, N_CLAUSES, unroll=2,
                                carry=jnp.full((16,), 2, jnp.int32))
            def second_loop(c, col):
                d0, d1, d2 = clause_deltas(c, col, col + 1, col + 2)
                plsc.addupdate_scatter(out_buf, [rows, col], d0)
                plsc.addupdate_scatter(out_buf, [rows, col + 1], d1)
                plsc.addupdate_scatter(out_buf, [rows, col + 2], d2)
                return col + 3

    start_in(0, 0)

    @pl.when(n_my > 1)
    def _():
        start_in(1, 1)

    def pair_body(ip, carry):
        for b in range(2):
            i = ip * 2 + b

            @pl.when(i < n_my)
            def _():
                wait_in(b)

                @pl.when(i >= 2)
                def _():
                    wait_out(b)

                compute(b)
                start_out(i, b)

                @pl.when(i + 2 < n_my)
                def _():
                    start_in(i + 2, b)

        return carry

    lax.fori_loop(0, (n_my + 1) // 2, pair_body, 0)

    @pl.when(n_my >= 1)
    def _():
        wait_out(0)

    @pl.when(n_my >= 2)
    def _():
        wait_out(1)


def kernel(ground_atoms, edge_weight, clause_weights):
    b, n_pred = ground_atoms.shape
    assert n_pred == N_PRED and b % CHUNK_ROWS == 0
    ew = edge_weight.reshape(-1)
    cw2 = jnp.broadcast_to(clause_weights[:, None], (N_CLAUSES, 16))
    mesh = plsc.VectorSubcoreMesh(
        core_axis_name="c", subcore_axis_name="s", num_cores=2, num_subcores=16
    )
    call = pl.kernel(
        _sc_body,
        out_type=jax.ShapeDtypeStruct((b, N_PRED), jnp.float32),
        mesh=mesh,
        scratch_types=[
            [pltpu.VMEM((CHUNK_ROWS, ROW_PITCH), jnp.float32) for _ in range(2)],
            [pltpu.VMEM((CHUNK_ROWS,), jnp.float32) for _ in range(2)],
            pltpu.VMEM((N_CLAUSES, 16), jnp.float32),
            [pltpu.VMEM((CHUNK_ROWS, ROW_PITCH), jnp.float32) for _ in range(2)],
            [pltpu.SemaphoreType.DMA for _ in range(2)],
            [pltpu.SemaphoreType.DMA for _ in range(2)],
        ],
        compiler_params=pltpu.CompilerParams(
            use_tc_tiling_on_sc=False, needs_layout_passes=False
        ),
    )
    return call(ground_atoms, ew, cw2)
